# baseline (speedup 1.0000x reference)
import functools

import jax
import jax.numpy as jnp
from jax import lax
from jax.experimental import pallas as pl
from jax.experimental.pallas import tpu as pltpu
from jax.experimental.pallas import tpu_sc as plsc

B = 4096
V = 6890
K = V * 3
NJ = 26
NCOL = NJ * 3
NIDX = 11
GN = B * NIDX * 3

SC_NC = 2
SC_NW = 32
CHUNK = 128
PER_W = GN // SC_NW
ROWS_W = PER_W // CHUNK


def _tc_body(m_ref, jexp_ref, out_ref):
    out_ref[...] = jnp.dot(m_ref[...], jexp_ref[...],
                           preferred_element_type=jnp.float32)


def _tc_regress(m, jexp, bb=128):
    return pl.pallas_call(
        _tc_body,
        grid=(B // bb,),
        in_specs=[
            pl.BlockSpec((bb, K), lambda i: (i, 0)),
            pl.BlockSpec((K, NCOL), lambda i: (0, 0)),
        ],
        out_specs=pl.BlockSpec((bb, NCOL), lambda i: (i, 0)),
        out_shape=jax.ShapeDtypeStruct((B, NCOL), jnp.float32),
    )(m, jexp)


@functools.cache
def _make_sc_gather():
    @functools.partial(
        pl.kernel,
        mesh=plsc.VectorSubcoreMesh(core_axis_name="c", subcore_axis_name="s"),
        out_type=jax.ShapeDtypeStruct((GN,), jnp.float32),
        scratch_types=[
            pltpu.VMEM((ROWS_W, CHUNK), jnp.int32),
            pltpu.VMEM((PER_W,), jnp.float32),
            pltpu.SemaphoreType.DMA,
        ],
    )
    def _sc_gather(vflat_hbm, eidx_hbm, out_hbm, idx_v, rows_v, sem):
        wid = lax.axis_index("s") * SC_NC + lax.axis_index("c")
        pltpu.sync_copy(eidx_hbm.at[wid], idx_v)

        def body(i, carry):
            pltpu.async_copy(vflat_hbm.at[idx_v.at[i]],
                             rows_v.at[pl.ds(i * CHUNK, CHUNK)], sem)
            return carry

        lax.fori_loop(0, ROWS_W, body, 0)
        pltpu.make_async_copy(vflat_hbm.at[pl.ds(0, PER_W)], rows_v,
                              sem).wait()
        pltpu.sync_copy(rows_v, out_hbm.at[pl.ds(wid * PER_W, PER_W)])

    return _sc_gather


def kernel(vertices, joints, extra_joints_idxs, J_regressor_extra9,
           J_regressor_h36m17):
    m = vertices.reshape(B, K)
    vflat = vertices.reshape(GN // (NIDX * 3) * V * 3)

    j26 = jnp.concatenate([J_regressor_extra9, J_regressor_h36m17], axis=0)
    eye3 = jnp.eye(3, dtype=jnp.float32)
    jexp = (j26.T[:, None, :, None] * eye3[None, :, None, :]).reshape(K, NCOL)

    base_b = (jnp.arange(B, dtype=jnp.int32) * (V * 3))[:, None, None]
    off = (extra_joints_idxs.astype(jnp.int32) * 3)[None, :, None] \
        + jnp.arange(3, dtype=jnp.int32)[None, None, :]
    eidx = (base_b + off).reshape(SC_NW, ROWS_W, CHUNK)

    extra_flat = _make_sc_gather()(vflat, eidx)
    reg = _tc_regress(m, jexp)

    out = jnp.concatenate(
        [joints.reshape(B, 72), extra_flat.reshape(B, NIDX * 3), reg], axis=1)
    return out.reshape(B, 61, 3)


# speedup vs baseline: 931.9473x; 931.9473x over previous
import jax
import jax.numpy as jnp
from jax.experimental import pallas as pl

B = 4096
V = 6890
NROWS = 37
BB = 512


def _body(j37_ref, vt_ref, jt_ref, out_ref):
    plane = vt_ref[0]
    reg = jnp.dot(j37_ref[...], plane,
                  preferred_element_type=jnp.float32)
    out_ref[0, :24, :] = jt_ref[0]
    out_ref[0, 24:, :] = reg


def kernel(vertices, joints, extra_joints_idxs, J_regressor_extra9,
           J_regressor_h36m17):
    vt = jnp.transpose(vertices, (2, 1, 0))
    jt = jnp.transpose(joints, (2, 1, 0))

    onehot = jax.nn.one_hot(extra_joints_idxs, V, dtype=jnp.float32)
    j37 = jnp.concatenate([onehot, J_regressor_extra9, J_regressor_h36m17],
                          axis=0)

    out_t = pl.pallas_call(
        _body,
        grid=(3, B // BB),
        in_specs=[
            pl.BlockSpec((NROWS, V), lambda k, b: (0, 0)),
            pl.BlockSpec((1, V, BB), lambda k, b: (k, 0, b)),
            pl.BlockSpec((1, 24, BB), lambda k, b: (k, 0, b)),
        ],
        out_specs=pl.BlockSpec((1, 61, BB), lambda k, b: (k, 0, b)),
        out_shape=jax.ShapeDtypeStruct((3, 61, B), jnp.float32),
    )(j37, vt, jt)

    return jnp.transpose(out_t, (2, 1, 0))
